# baseline (speedup 1.0000x reference)
import functools

import jax
import jax.numpy as jnp
from jax import lax
from jax.experimental import pallas as pl
from jax.experimental.pallas import tpu as pltpu
from jax.experimental.pallas import tpu_sc as plsc

B, N, F_IN = 8, 2048, 128
C, D, H, W_DIM = 128, 32, 32, 32
DHW = D * H * W_DIM

NB = 256
NUM_NB = N // NB

NW = 32
PTS = B * N
PW = PTS // NW
CHUNK = 16
NCH = PW // CHUNK


def _pos_body(adj_ref, x_ref, w_ref, pos_ref, w8_ref, idx8_ref):
    b = pl.program_id(0)
    h = jnp.dot(adj_ref[0].astype(jnp.bfloat16),
                x_ref[0].astype(jnp.bfloat16),
                preferred_element_type=jnp.float32)
    pos = jnp.dot(h.astype(jnp.bfloat16),
                  w_ref[...].astype(jnp.bfloat16),
                  preferred_element_type=jnp.float32)
    pos_ref[0] = pos
    pos_g = 2.0 * pos - 1.0
    ix = (pos_g[:, 0:1] + 1.0) * ((W_DIM - 1) / 2.0)
    iy = (pos_g[:, 1:2] + 1.0) * ((H - 1) / 2.0)
    iz = (pos_g[:, 2:3] + 1.0) * ((D - 1) / 2.0)
    x0 = jnp.floor(ix)
    y0 = jnp.floor(iy)
    z0 = jnp.floor(iz)
    ws = []
    idxs = []
    for dz in (0, 1):
        for dy in (0, 1):
            for dx in (0, 1):
                xc = x0 + dx
                yc = y0 + dy
                zc = z0 + dz
                w = ((1.0 - jnp.abs(ix - xc))
                     * (1.0 - jnp.abs(iy - yc))
                     * (1.0 - jnp.abs(iz - zc)))
                inb = ((xc >= 0) & (xc <= W_DIM - 1)
                       & (yc >= 0) & (yc <= H - 1)
                       & (zc >= 0) & (zc <= D - 1))
                xi = jnp.clip(xc, 0, W_DIM - 1).astype(jnp.int32)
                yi = jnp.clip(yc, 0, H - 1).astype(jnp.int32)
                zi = jnp.clip(zc, 0, D - 1).astype(jnp.int32)
                flat = ((b * D + zi) * H + yi) * W_DIM + xi
                ws.append(jnp.broadcast_to(w * inb.astype(jnp.float32),
                                           (NB, 16)))
                idxs.append(flat)
    w8_ref[0] = jnp.concatenate(ws, axis=1)
    idx8_ref[0] = jnp.concatenate(idxs, axis=1)


def _pos_call(adj, x, W):
    return pl.pallas_call(
        _pos_body,
        grid=(B, NUM_NB),
        in_specs=[
            pl.BlockSpec((1, NB, N), lambda b, nb: (b, nb, 0)),
            pl.BlockSpec((1, N, F_IN), lambda b, nb: (b, 0, 0)),
            pl.BlockSpec((F_IN, 3), lambda b, nb: (0, 0)),
        ],
        out_specs=[
            pl.BlockSpec((1, NB, 3), lambda b, nb: (b, nb, 0)),
            pl.BlockSpec((1, NB, 128), lambda b, nb: (b, nb, 0)),
            pl.BlockSpec((1, NB, 8), lambda b, nb: (b, nb, 0)),
        ],
        out_shape=[
            jax.ShapeDtypeStruct((B, N, 3), jnp.float32),
            jax.ShapeDtypeStruct((B, N, 128), jnp.float32),
            jax.ShapeDtypeStruct((B, N, 8), jnp.int32),
        ],
    )(adj, x, W)


TB = 2048


def _tr_body(v_ref, out_ref):
    out_ref[0] = v_ref[0].T


def _tr_call(vol2d):
    return pl.pallas_call(
        _tr_body,
        grid=(B, DHW // TB),
        in_specs=[pl.BlockSpec((1, C, TB), lambda b, t: (b, 0, t))],
        out_specs=pl.BlockSpec((1, TB, C), lambda b, t: (b, t, 0)),
        out_shape=jax.ShapeDtypeStruct((B, DHW, C), jnp.float32),
    )(vol2d)


NUM_CORES = 2
NUM_SUBCORES = 16


def _sc_body(table_hbm, idx_hbm, w_hbm, out_hbm, idx_v, w_v, rows_v,
             out_v, sem):
    wid = lax.axis_index("s") * NUM_CORES + lax.axis_index("c")
    base = wid * PW
    pltpu.sync_copy(w_hbm.at[pl.ds(base, PW)], w_v)

    def chunk_body(ch, carry):
        cbase = base * 8 + ch * (CHUNK * 8)
        pltpu.sync_copy(idx_hbm.at[pl.ds(cbase, CHUNK * 8)], idx_v)
        pltpu.async_copy(table_hbm.at[idx_v], rows_v, sem).wait()
        for p in range(CHUNK):
            pt = ch * CHUNK + p
            acc = [jnp.zeros((16,), jnp.float32) for _ in range(8)]
            for k in range(8):
                r = p * 8 + k
                wk = w_v[pt, pl.ds(k * 16, 16)]
                for v in range(8):
                    acc[v] = acc[v] + wk * rows_v[r, pl.ds(v * 16, 16)]
            for v in range(8):
                out_v[p, pl.ds(v * 16, 16)] = acc[v]
        pltpu.sync_copy(out_v, out_hbm.at[pl.ds(base + ch * CHUNK, CHUNK)])
        return carry

    lax.fori_loop(0, NCH, chunk_body, 0)


@functools.cache
def _sc_gather():
    mesh = plsc.VectorSubcoreMesh(
        core_axis_name="c", subcore_axis_name="s",
        num_cores=NUM_CORES, num_subcores=NUM_SUBCORES)
    return pl.kernel(
        _sc_body,
        mesh=mesh,
        out_type=jax.ShapeDtypeStruct((PTS, C), jnp.float32),
        scratch_types=[
            pltpu.VMEM((CHUNK * 8,), jnp.int32),
            pltpu.VMEM((PW, 8 * 16), jnp.float32),
            pltpu.VMEM((CHUNK * 8, C), jnp.float32),
            pltpu.VMEM((CHUNK, C), jnp.float32),
            pltpu.SemaphoreType.DMA,
        ],
    )


def kernel(x, adj, conv_layer, W):
    pos, w8, idx8 = _pos_call(adj, x, W)
    table = _tr_call(conv_layer.reshape(B, C, DHW))
    skip = _sc_gather()(table.reshape(B * DHW, C),
                        idx8.reshape(PTS * 8), w8.reshape(PTS, 8 * 16))
    x_out = jnp.concatenate([x, skip.reshape(B, N, C), pos], axis=2)
    return (x_out, pos)


# speedup vs baseline: 1.1727x; 1.1727x over previous
import functools

import jax
import jax.numpy as jnp
from jax import lax
from jax.experimental import pallas as pl
from jax.experimental.pallas import tpu as pltpu
from jax.experimental.pallas import tpu_sc as plsc

B, N, F_IN = 8, 2048, 128
C, D, H, W_DIM = 128, 32, 32, 32
DHW = D * H * W_DIM

NB = 512
NUM_NB = N // NB

NW = 32
PTS = B * N
PW = PTS // NW
CHUNK = 16
NCH = PW // CHUNK
NBUF = 2

NUM_CORES = 2
NUM_SUBCORES = 16


def _pos_body(adj_ref, x_ref, w_ref, pos_ref, w8_ref, idx8_ref):
    b = pl.program_id(0)
    h = jnp.dot(adj_ref[0].astype(jnp.bfloat16),
                x_ref[0].astype(jnp.bfloat16),
                preferred_element_type=jnp.float32)
    pos = jnp.dot(h.astype(jnp.bfloat16),
                  w_ref[...].astype(jnp.bfloat16),
                  preferred_element_type=jnp.float32)
    pos_ref[0] = pos

    shape = (NB, 128)
    coords = []
    for a in range(3):
        p = jnp.broadcast_to(pos[:, a:a + 1], shape)
        pg = 2.0 * p - 1.0
        i = (pg + 1.0) * ((W_DIM - 1) / 2.0)
        i0 = jnp.floor(i)
        coords.append((i, i0))
    lane = lax.broadcasted_iota(jnp.int32, shape, 1)

    def corner_terms(k_of_lane):
        ws, inb, ci = [], None, []
        for a, (i, i0) in enumerate(coords):
            d = ((k_of_lane >> a) & 1).astype(jnp.float32)
            cc = i0 + d
            ws.append(1.0 - jnp.abs(i - cc))
            ok = (cc >= 0) & (cc <= W_DIM - 1)
            inb = ok if inb is None else (inb & ok)
            ci.append(jnp.clip(cc, 0, W_DIM - 1).astype(jnp.int32))
        return ws, inb, ci

    ws, inb, _ = corner_terms(lane >> 4)
    w8_ref[0] = ws[0] * ws[1] * ws[2] * inb.astype(jnp.float32)

    _, _, ci = corner_terms(lane)
    flat = ((b * D + ci[2]) * H + ci[1]) * W_DIM + ci[0]
    idx8_ref[0] = flat[:, 0:8]


def _pos_call(adj, x, W):
    return pl.pallas_call(
        _pos_body,
        grid=(B, NUM_NB),
        in_specs=[
            pl.BlockSpec((1, NB, N), lambda b, nb: (b, nb, 0)),
            pl.BlockSpec((1, N, F_IN), lambda b, nb: (b, 0, 0)),
            pl.BlockSpec((F_IN, 3), lambda b, nb: (0, 0)),
        ],
        out_specs=[
            pl.BlockSpec((1, NB, 3), lambda b, nb: (b, nb, 0)),
            pl.BlockSpec((1, NB, 128), lambda b, nb: (b, nb, 0)),
            pl.BlockSpec((1, NB, 8), lambda b, nb: (b, nb, 0)),
        ],
        out_shape=[
            jax.ShapeDtypeStruct((B, N, 3), jnp.float32),
            jax.ShapeDtypeStruct((B, N, 128), jnp.float32),
            jax.ShapeDtypeStruct((B, N, 8), jnp.int32),
        ],
    )(adj, x, W)


TB = 4096


def _tr_body(v_ref, out_ref):
    out_ref[0] = v_ref[0].T


def _tr_call(vol2d):
    return pl.pallas_call(
        _tr_body,
        grid=(B, DHW // TB),
        in_specs=[pl.BlockSpec((1, C, TB), lambda b, t: (b, 0, t))],
        out_specs=pl.BlockSpec((1, TB, C), lambda b, t: (b, t, 0)),
        out_shape=jax.ShapeDtypeStruct((B, DHW, C), jnp.float32),
    )(vol2d)


def _sc_body(table_hbm, idx_hbm, w_hbm, out_hbm, idx_v, w_v, rows_v,
             out_v, *sems):
    gsem = sems[0:NBUF]
    wsem = sems[NBUF:2 * NBUF]
    osem = sems[2 * NBUF:3 * NBUF]
    wid = lax.axis_index("s") * NUM_CORES + lax.axis_index("c")
    base = wid * PW
    pltpu.sync_copy(idx_hbm.at[wid], idx_v)

    def start_chunk(ch, b):
        pltpu.async_copy(table_hbm.at[idx_v.at[ch]], rows_v.at[b], gsem[b])
        pltpu.async_copy(w_hbm.at[pl.ds(base + ch * CHUNK, CHUNK)],
                         w_v.at[b], wsem[b])

    for b in range(NBUF):
        start_chunk(b, b)

    def group(g, carry):
        for b in range(NBUF):
            ch = g * NBUF + b
            pltpu.make_async_copy(table_hbm.at[idx_v.at[b]], rows_v.at[b],
                                  gsem[b]).wait()
            pltpu.make_async_copy(w_hbm.at[pl.ds(base, CHUNK)], w_v.at[b],
                                  wsem[b]).wait()

            @pl.when(g > 0)
            def _():
                pltpu.make_async_copy(out_v.at[b],
                                      out_hbm.at[pl.ds(base, CHUNK)],
                                      osem[b]).wait()

            for p in range(CHUNK):
                acc = [None] * 8
                for k in range(8):
                    r = p * 8 + k
                    wk = w_v[b, p, pl.ds(k * 16, 16)]
                    for v in range(8):
                        t = wk * rows_v[b, r, pl.ds(v * 16, 16)]
                        acc[v] = t if acc[v] is None else acc[v] + t
                for v in range(8):
                    out_v[b, p, pl.ds(v * 16, 16)] = acc[v]

            @pl.when(ch + NBUF < NCH)
            def _():
                start_chunk(ch + NBUF, b)

            pltpu.async_copy(out_v.at[b],
                             out_hbm.at[pl.ds(base + ch * CHUNK, CHUNK)],
                             osem[b])
        return carry

    lax.fori_loop(0, NCH // NBUF, group, 0)
    for b in range(NBUF):
        pltpu.make_async_copy(out_v.at[b], out_hbm.at[pl.ds(base, CHUNK)],
                              osem[b]).wait()


@functools.cache
def _sc_gather():
    mesh = plsc.VectorSubcoreMesh(
        core_axis_name="c", subcore_axis_name="s",
        num_cores=NUM_CORES, num_subcores=NUM_SUBCORES)
    return pl.kernel(
        _sc_body,
        mesh=mesh,
        out_type=jax.ShapeDtypeStruct((PTS, C), jnp.float32),
        scratch_types=[
            pltpu.VMEM((NCH, CHUNK * 8), jnp.int32),
            pltpu.VMEM((NBUF, CHUNK, 8 * 16), jnp.float32),
            pltpu.VMEM((NBUF, CHUNK * 8, C), jnp.float32),
            pltpu.VMEM((NBUF, CHUNK, C), jnp.float32),
        ] + [pltpu.SemaphoreType.DMA] * (3 * NBUF),
    )


def kernel(x, adj, conv_layer, W):
    pos, w8, idx8 = _pos_call(adj, x, W)
    table = _tr_call(conv_layer.reshape(B, C, DHW))
    skip = _sc_gather()(table.reshape(B * DHW, C),
                        idx8.reshape(NW, NCH, CHUNK * 8),
                        w8.reshape(PTS, 8 * 16))
    x_out = jnp.concatenate([x, skip.reshape(B, N, C), pos], axis=2)
    return (x_out, pos)


# speedup vs baseline: 1.8643x; 1.5897x over previous
import functools

import jax
import jax.numpy as jnp
from jax import lax
from jax.experimental import pallas as pl
from jax.experimental.pallas import tpu as pltpu
from jax.experimental.pallas import tpu_sc as plsc

B, N, F_IN = 8, 2048, 128
C, D, H, W_DIM = 128, 32, 32, 32
DHW = D * H * W_DIM

NB = 512
NUM_NB = N // NB

NW = 32
PTS = B * N
PW = PTS // NW
CHUNK = 16
NCH = PW // CHUNK
NBUF = 4

NUM_CORES = 2
NUM_SUBCORES = 16


def _pos_body(adj_ref, x_ref, w_ref, pos_ref, w8_ref, idx8_ref):
    b = pl.program_id(0)
    h = jnp.dot(adj_ref[0].astype(jnp.bfloat16),
                x_ref[0].astype(jnp.bfloat16),
                preferred_element_type=jnp.float32)
    pos = jnp.dot(h.astype(jnp.bfloat16),
                  w_ref[...].astype(jnp.bfloat16),
                  preferred_element_type=jnp.float32)
    pos_ref[0] = pos

    shape = (NB, 128)
    coords = []
    for a in range(3):
        p = jnp.broadcast_to(pos[:, a:a + 1], shape)
        pg = 2.0 * p - 1.0
        i = (pg + 1.0) * ((W_DIM - 1) / 2.0)
        i0 = jnp.floor(i)
        coords.append((i, i0))
    lane = lax.broadcasted_iota(jnp.int32, shape, 1)

    def corner_terms(k_of_lane):
        ws, inb, ci = [], None, []
        for a, (i, i0) in enumerate(coords):
            d = ((k_of_lane >> a) & 1).astype(jnp.float32)
            cc = i0 + d
            ws.append(1.0 - jnp.abs(i - cc))
            ok = (cc >= 0) & (cc <= W_DIM - 1)
            inb = ok if inb is None else (inb & ok)
            ci.append(jnp.clip(cc, 0, W_DIM - 1).astype(jnp.int32))
        return ws, inb, ci

    ws, inb, _ = corner_terms(lane >> 4)
    w8_ref[0] = ws[0] * ws[1] * ws[2] * inb.astype(jnp.float32)

    _, _, ci = corner_terms(lane)
    flat = ((b * D + ci[2]) * H + ci[1]) * W_DIM + ci[0]
    idx8_ref[0] = flat[:, 0:8]


def _pos_call(adj, x, W):
    return pl.pallas_call(
        _pos_body,
        grid=(B, NUM_NB),
        in_specs=[
            pl.BlockSpec((1, NB, N), lambda b, nb: (b, nb, 0)),
            pl.BlockSpec((1, N, F_IN), lambda b, nb: (b, 0, 0)),
            pl.BlockSpec((F_IN, 3), lambda b, nb: (0, 0)),
        ],
        out_specs=[
            pl.BlockSpec((1, NB, 3), lambda b, nb: (b, nb, 0)),
            pl.BlockSpec((1, NB, 128), lambda b, nb: (b, nb, 0)),
            pl.BlockSpec((1, NB, 8), lambda b, nb: (b, nb, 0)),
        ],
        out_shape=[
            jax.ShapeDtypeStruct((B, N, 3), jnp.float32),
            jax.ShapeDtypeStruct((B, N, 128), jnp.float32),
            jax.ShapeDtypeStruct((B, N, 8), jnp.int32),
        ],
    )(adj, x, W)


def _sc_body(table_hbm, idx_hbm, w_hbm, out_hbm, idx_v, w_v, rows_v,
             out_v, *sems):
    gsem = sems[0:NBUF]
    wsem = sems[NBUF:2 * NBUF]
    osem = sems[2 * NBUF:3 * NBUF]
    wid = lax.axis_index("s") * NUM_CORES + lax.axis_index("c")
    base = wid * PW
    pltpu.sync_copy(idx_hbm.at[wid], idx_v)

    def start_chunk(ch, b):
        pltpu.async_copy(table_hbm.at[idx_v.at[ch]], rows_v.at[b], gsem[b])
        pltpu.async_copy(w_hbm.at[pl.ds(base + ch * CHUNK, CHUNK)],
                         w_v.at[b], wsem[b])

    for b in range(NBUF):
        start_chunk(b, b)

    def group(g, carry):
        for b in range(NBUF):
            ch = g * NBUF + b
            pltpu.make_async_copy(table_hbm.at[idx_v.at[b]], rows_v.at[b],
                                  gsem[b]).wait()
            pltpu.make_async_copy(w_hbm.at[pl.ds(base, CHUNK)], w_v.at[b],
                                  wsem[b]).wait()

            @pl.when(g > 0)
            def _():
                pltpu.make_async_copy(out_v.at[b],
                                      out_hbm.at[pl.ds(base, CHUNK)],
                                      osem[b]).wait()

            def point_body(p, pcarry):
                wks = [w_v[b, p, pl.ds(k * 16, 16)] for k in range(8)]
                for v in range(8):
                    ts = [wks[k] * rows_v[b, p * 8 + k, pl.ds(v * 16, 16)]
                          for k in range(8)]
                    while len(ts) > 1:
                        ts = [ts[i] + ts[i + 1] for i in range(0, len(ts), 2)]
                    out_v[b, p, pl.ds(v * 16, 16)] = ts[0]
                return pcarry

            lax.fori_loop(0, CHUNK, point_body, 0)

            @pl.when(ch + NBUF < NCH)
            def _():
                start_chunk(ch + NBUF, b)

            pltpu.async_copy(out_v.at[b],
                             out_hbm.at[pl.ds(base + ch * CHUNK, CHUNK)],
                             osem[b])
        return carry

    lax.fori_loop(0, NCH // NBUF, group, 0)
    for b in range(NBUF):
        pltpu.make_async_copy(out_v.at[b], out_hbm.at[pl.ds(base, CHUNK)],
                              osem[b]).wait()


@functools.cache
def _sc_gather():
    mesh = plsc.VectorSubcoreMesh(
        core_axis_name="c", subcore_axis_name="s",
        num_cores=NUM_CORES, num_subcores=NUM_SUBCORES)
    return pl.kernel(
        _sc_body,
        mesh=mesh,
        out_type=jax.ShapeDtypeStruct((PTS, C), jnp.float32),
        scratch_types=[
            pltpu.VMEM((NCH, CHUNK * 8), jnp.int32),
            pltpu.VMEM((NBUF, CHUNK, 8 * 16), jnp.float32),
            pltpu.VMEM((NBUF, CHUNK * 8, C), jnp.float32),
            pltpu.VMEM((NBUF, CHUNK, C), jnp.float32),
        ] + [pltpu.SemaphoreType.DMA] * (3 * NBUF),
    )


def kernel(x, adj, conv_layer, W):
    pos, w8, idx8 = _pos_call(adj, x, W)
    table = jnp.transpose(conv_layer.reshape(B, C, DHW), (0, 2, 1))
    skip = _sc_gather()(table.reshape(B * DHW, C),
                        idx8.reshape(NW, NCH, CHUNK * 8),
                        w8.reshape(PTS, 8 * 16))
    x_out = jnp.concatenate([x, skip.reshape(B, N, C), pos], axis=2)
    return (x_out, pos)
